# Initial kernel scaffold; baseline (speedup 1.0000x reference)
import functools

import jax
import jax.numpy as jnp
from jax import lax
from jax.experimental import pallas as pl
from jax.experimental.pallas import tpu as pltpu
from jax.experimental.pallas import tpu_sc as plsc

F32 = jnp.float32
N_NODE = 10000
HID = 128
CHUNK = 128
NWORK = 32
NODE_BLK = 2000
EDGE_BLK = 2000


def _tables_body(pos_s_ref, h_ref, x_ref, pos_a_ref, u_ref,
                 wp_ref, wh_ref, wx_ref, bs_ref, wpa_ref, wua_ref,
                 tadst_ref, tssrc_ref, tsdst_ref, tasrc_ref):
    pos_s = pos_s_ref[...]
    wp = wp_ref[...]
    r = (pos_s[:, 0:1] * wp[0:1, :] + pos_s[:, 1:2] * wp[1:2, :]
         + jnp.dot(h_ref[...], wh_ref[...], preferred_element_type=F32)
         + jnp.dot(x_ref[...], wx_ref[...], preferred_element_type=F32)
         + bs_ref[...])
    tadst_ref[...] = r[:, 0:HID]
    tssrc_ref[...] = r[:, HID:2 * HID]
    tsdst_ref[...] = r[:, 2 * HID:3 * HID]
    pos_a = pos_a_ref[...]
    wpa = wpa_ref[...]
    tasrc_ref[...] = (pos_a[:, 0:1] * wpa[0:1, :] + pos_a[:, 1:2] * wpa[1:2, :]
                      + jnp.dot(u_ref[...], wua_ref[...], preferred_element_type=F32))


def _node_tables(pos_s, h, x, pos_a, u, wp, wh, wx, bs, wpa, wua):
    n = pos_s.shape[0]
    grid = (n // NODE_BLK,)
    row = lambda w: pl.BlockSpec((NODE_BLK, w), lambda i: (i, 0))
    full = lambda a, b: pl.BlockSpec((a, b), lambda i: (0, 0))
    return pl.pallas_call(
        _tables_body,
        grid=grid,
        in_specs=[row(2), row(HID), row(HID), row(2), row(HID),
                  full(2, 3 * HID), full(HID, 3 * HID), full(HID, 3 * HID),
                  full(1, 3 * HID), full(2, HID), full(HID, HID)],
        out_specs=[row(HID), row(HID), row(HID), row(HID)],
        out_shape=[jax.ShapeDtypeStruct((n, HID), F32)] * 4,
    )(pos_s, h, x, pos_a, u, wp, wh, wx, bs, wpa, wua)


def _gather_body(si_a, di_a, si_s, di_s, ta_s, ta_d, ts_s, ts_d,
                 g0, g1, g2, g3,
                 i0, i1, i2, i3, r0, r1, r2, r3, s0, s1, s2, s3):
    c = lax.axis_index("c")
    s = lax.axis_index("s")
    wid = s * 2 + c
    nch_total = g0.shape[0] // CHUNK
    extra = nch_total % NWORK
    nch = nch_total // NWORK + jnp.where(wid < extra, 1, 0)
    idx_hbm = (si_a, di_a, si_s, di_s)
    tabs = (ta_s, ta_d, ts_s, ts_d)
    outs = (g0, g1, g2, g3)
    idxv = (i0, i1, i2, i3)
    rowv = (r0, r1, r2, r3)
    sems = (s0, s1, s2, s3)

    def body(i, carry):
        base = (wid + i * NWORK) * CHUNK
        for k in range(4):
            pltpu.sync_copy(idx_hbm[k].at[pl.ds(base, CHUNK)], idxv[k])
        cps = [pltpu.async_copy(tabs[k].at[idxv[k]], rowv[k], sems[k])
               for k in range(4)]
        for cp in cps:
            cp.wait()
        for k in range(4):
            pltpu.sync_copy(rowv[k], outs[k].at[pl.ds(base, CHUNK)])
        return carry

    lax.fori_loop(0, nch, body, 0)


def _gather_tables(si_a, di_a, si_s, di_s, ta_s, ta_d, ts_s, ts_d):
    ea = si_a.shape[0]
    mesh = plsc.VectorSubcoreMesh(core_axis_name="c", subcore_axis_name="s")
    scratch = ([pltpu.VMEM((CHUNK,), jnp.int32)] * 4
               + [pltpu.VMEM((CHUNK, HID), F32)] * 4
               + [pltpu.SemaphoreType.DMA] * 4)
    fn = pl.kernel(
        _gather_body,
        out_type=[jax.ShapeDtypeStruct((ea, HID), F32)] * 4,
        mesh=mesh,
        scratch_types=scratch,
    )
    return fn(si_a, di_a, si_s, di_s, ta_s, ta_d, ts_s, ts_d)


def _edge_body(gs_ref, gd_ref, dis_ref, wd_ref, w2_ref, b2_ref, w3_ref, b3_ref,
               out_ref):
    z1 = jnp.tanh(gs_ref[...] + gd_ref[...] + dis_ref[...] * wd_ref[...])
    h2 = jnp.tanh(jnp.dot(z1, w2_ref[...], preferred_element_type=F32)
                  + b2_ref[...])
    o = jnp.dot(h2, w3_ref[...], preferred_element_type=F32) + b3_ref[...]
    el = jnp.exp(o[:, 0:HID])
    out_ref[...] = jnp.concatenate([el * o[:, HID:2 * HID], el], axis=1)


def _edge_mlp(gs, gd, dis, wd, w2, b2, w3, b3):
    ea = gs.shape[0]
    grid = (ea // EDGE_BLK,)
    row = lambda w: pl.BlockSpec((EDGE_BLK, w), lambda i: (i, 0))
    full = lambda a, b: pl.BlockSpec((a, b), lambda i: (0, 0))
    return pl.pallas_call(
        _edge_body,
        grid=grid,
        in_specs=[row(HID), row(HID), row(1),
                  full(1, HID), full(HID, HID), full(1, HID),
                  full(HID, 2 * HID), full(1, 2 * HID)],
        out_specs=row(2 * HID),
        out_shape=jax.ShapeDtypeStruct((ea, 2 * HID), F32),
    )(gs, gd, dis, wd, w2, b2, w3, b3)


def _scatter_body(didx, contrib, zeros, out, idxv, bufv, acc_sh, sem):
    c = lax.axis_index("c")
    s = lax.axis_index("s")
    wid = s * 2 + c
    nch_total = contrib.shape[0] // CHUNK
    extra = nch_total % NWORK
    nch = nch_total // NWORK + jnp.where(wid < extra, 1, 0)

    @pl.when(s == 0)
    def _():
        pltpu.sync_copy(zeros, acc_sh)

    plsc.subcore_barrier()

    def body(i, carry):
        base = (wid + i * NWORK) * CHUNK
        pltpu.sync_copy(didx.at[pl.ds(base, CHUNK)], idxv)
        pltpu.sync_copy(contrib.at[pl.ds(base, CHUNK), pl.ds(c * HID, HID)],
                        bufv)
        pltpu.sync_copy(bufv, acc_sh.at[idxv], add=True)
        return carry

    lax.fori_loop(0, nch, body, 0)
    plsc.subcore_barrier()

    @pl.when(s < 10)
    def _():
        r0 = s * 1000
        pltpu.sync_copy(acc_sh.at[pl.ds(r0, 1000)],
                        out.at[pl.ds(r0, 1000), pl.ds(c * HID, HID)])


def _scatter_add(didx, contrib, zeros):
    mesh = plsc.VectorSubcoreMesh(core_axis_name="c", subcore_axis_name="s")
    scratch = [pltpu.VMEM((CHUNK,), jnp.int32),
               pltpu.VMEM((CHUNK, HID), F32),
               pltpu.VMEM_SHARED((N_NODE, HID), F32),
               pltpu.SemaphoreType.DMA]
    fn = pl.kernel(
        _scatter_body,
        out_type=jax.ShapeDtypeStruct((N_NODE, 2 * HID), F32),
        mesh=mesh,
        scratch_types=scratch,
    )
    return fn(didx, contrib, zeros)


def _final_body(pos_ref, h_ref, x_ref, acca_ref, accs_ref,
                wp_ref, wh_ref, wsu_ref, wsx_ref, wx2_ref, b1_ref,
                w2_ref, b2_ref, w3_ref, b3_ref, out_ref):
    acca = acca_ref[...]
    accs = accs_ref[...]
    dena = acca[:, HID:2 * HID]
    dens = accs[:, HID:2 * HID]
    sum_u = jnp.where(dena != 0, acca[:, 0:HID] / dena, 0.0)
    sum_x = jnp.where(dens != 0, accs[:, 0:HID] / dens, 0.0)
    pos = pos_ref[...]
    wp = wp_ref[...]
    t1 = jnp.tanh(
        pos[:, 0:1] * wp[0:1, :] + pos[:, 1:2] * wp[1:2, :]
        + jnp.dot(h_ref[...], wh_ref[...], preferred_element_type=F32)
        + jnp.dot(sum_u, wsu_ref[...], preferred_element_type=F32)
        + jnp.dot(sum_x, wsx_ref[...], preferred_element_type=F32)
        + jnp.dot(x_ref[...], wx2_ref[...], preferred_element_type=F32)
        + b1_ref[...])
    t2 = jnp.tanh(jnp.dot(t1, w2_ref[...], preferred_element_type=F32)
                  + b2_ref[...])
    out_ref[...] = (jnp.dot(t2, w3_ref[...], preferred_element_type=F32)
                    + b3_ref[...])


def _final_mlp(pos_s, h, x, acca, accs, wp, wh, wsu, wsx, wx2, b1, w2, b2, w3,
               b3):
    n = pos_s.shape[0]
    grid = (n // NODE_BLK,)
    row = lambda w: pl.BlockSpec((NODE_BLK, w), lambda i: (i, 0))
    full = lambda a, b: pl.BlockSpec((a, b), lambda i: (0, 0))
    mlp = 64
    return pl.pallas_call(
        _final_body,
        grid=grid,
        in_specs=[row(2), row(HID), row(HID), row(2 * HID), row(2 * HID),
                  full(2, mlp), full(HID, mlp), full(HID, mlp),
                  full(HID, mlp), full(HID, mlp), full(1, mlp),
                  full(mlp, mlp), full(1, mlp), full(mlp, HID),
                  full(1, HID)],
        out_specs=row(HID),
        out_shape=jax.ShapeDtypeStruct((n, HID), F32),
    )(pos_s, h, x, acca, accs, wp, wh, wsu, wsx, wx2, b1, w2, b2, w3, b3)


def _fuse_heads(pa, pb):
    w1 = jnp.concatenate([pa["W1"], pb["W1"]], axis=1)
    b1 = jnp.concatenate([pa["b1"], pb["b1"]])
    z = jnp.zeros_like(pa["W2"])
    w2 = jnp.concatenate([jnp.concatenate([pa["W2"], z], 1),
                          jnp.concatenate([z, pb["W2"]], 1)], 0)
    b2 = jnp.concatenate([pa["b2"], pb["b2"]])
    z3 = jnp.zeros_like(pa["W3"])
    w3 = jnp.concatenate([jnp.concatenate([pa["W3"], z3], 1),
                          jnp.concatenate([z3, pb["W3"]], 1)], 0)
    b3 = jnp.concatenate([pa["b3"], pb["b3"]])
    return w1, b1, w2, b2, w3, b3


def kernel(h, x, u, pos_state, pos_action, dis_a2s, dis_s2s, edge_a2s,
           edge_s2s, params):
    f = HID
    w1u, b1u, w2u, b2u, w3u, b3u = _fuse_heads(params["u2h_logit"],
                                               params["u2h_u"])
    w1x, b1x, w2x, b2x, w3x, b3x = _fuse_heads(params["x2h_logit"],
                                               params["x2h_x"])
    wp = jnp.concatenate([w1u[2:4], w1x[0:2], w1x[2:4]], axis=1)
    wh = jnp.concatenate([w1u[133:261], w1x[5:133], w1x[261:389]], 1)
    wx = jnp.concatenate([w1u[261:389], w1x[133:261], w1x[389:517]], 1)
    bs = jnp.concatenate([b1u, jnp.zeros_like(b1x), b1x]).reshape(1, 3 * f)
    wpa = w1u[0:2]
    wua = w1u[5:133]
    ta_dst, ts_src, ts_dst, ta_src = _node_tables(
        pos_state, h, x, pos_action, u, wp, wh, wx, bs, wpa, wua)

    src_a = edge_a2s[0].astype(jnp.int32)
    dst_a = edge_a2s[1].astype(jnp.int32)
    src_s = edge_s2s[0].astype(jnp.int32)
    dst_s = edge_s2s[1].astype(jnp.int32)

    g_as, g_ad, g_ss, g_sd = _gather_tables(
        src_a, dst_a, src_s, dst_s, ta_src, ta_dst, ts_src, ts_dst)

    contrib_a = _edge_mlp(g_as, g_ad, dis_a2s, w1u[4:5], w2u,
                          b2u.reshape(1, 2 * 64), w3u, b3u.reshape(1, 2 * f))
    contrib_s = _edge_mlp(g_ss, g_sd, dis_s2s, w1x[4:5], w2x,
                          b2x.reshape(1, 2 * 64), w3x, b3x.reshape(1, 2 * f))

    zeros = jnp.zeros((N_NODE, f), F32)
    acc_a = _scatter_add(dst_a, contrib_a, zeros)
    acc_s = _scatter_add(dst_s, contrib_s, zeros)

    pu = params["h_updater"]
    w1f = pu["W1"]
    return _final_mlp(
        pos_state, h, x, acc_a, acc_s,
        w1f[0:2], w1f[2:130], w1f[130:258], w1f[258:386], w1f[386:514],
        pu["b1"].reshape(1, -1), pu["W2"], pu["b2"].reshape(1, -1),
        pu["W3"], pu["b3"].reshape(1, -1))


# speedup vs baseline: 7.1790x; 7.1790x over previous
import functools

import jax
import jax.numpy as jnp
from jax import lax
from jax.experimental import pallas as pl
from jax.experimental.pallas import tpu as pltpu
from jax.experimental.pallas import tpu_sc as plsc

F32 = jnp.float32
N_NODE = 10000
HID = 128
CHUNK = 128
NWORK = 32
NODE_BLK = 2000
EDGE_BLK = 2000


def _tables_body(pos_s_ref, h_ref, x_ref, pos_a_ref, u_ref,
                 wp_ref, wh_ref, wx_ref, bs_ref, wpa_ref, wua_ref,
                 tadst_ref, tssrc_ref, tsdst_ref, tasrc_ref):
    pos_s = pos_s_ref[...]
    wp = wp_ref[...]
    r = (pos_s[:, 0:1] * wp[0:1, :] + pos_s[:, 1:2] * wp[1:2, :]
         + jnp.dot(h_ref[...], wh_ref[...], preferred_element_type=F32)
         + jnp.dot(x_ref[...], wx_ref[...], preferred_element_type=F32)
         + bs_ref[...])
    tadst_ref[...] = r[:, 0:HID]
    tssrc_ref[...] = r[:, HID:2 * HID]
    tsdst_ref[...] = r[:, 2 * HID:3 * HID]
    pos_a = pos_a_ref[...]
    wpa = wpa_ref[...]
    tasrc_ref[...] = (pos_a[:, 0:1] * wpa[0:1, :] + pos_a[:, 1:2] * wpa[1:2, :]
                      + jnp.dot(u_ref[...], wua_ref[...], preferred_element_type=F32))


def _node_tables(pos_s, h, x, pos_a, u, wp, wh, wx, bs, wpa, wua):
    n = pos_s.shape[0]
    grid = (n // NODE_BLK,)
    row = lambda w: pl.BlockSpec((NODE_BLK, w), lambda i: (i, 0))
    full = lambda a, b: pl.BlockSpec((a, b), lambda i: (0, 0))
    return pl.pallas_call(
        _tables_body,
        grid=grid,
        in_specs=[row(2), row(HID), row(HID), row(2), row(HID),
                  full(2, 3 * HID), full(HID, 3 * HID), full(HID, 3 * HID),
                  full(1, 3 * HID), full(2, HID), full(HID, HID)],
        out_specs=[row(HID), row(HID), row(HID), row(HID)],
        out_shape=[jax.ShapeDtypeStruct((n, HID), F32)] * 4,
    )(pos_s, h, x, pos_a, u, wp, wh, wx, bs, wpa, wua)


def _gather_body(si_a, di_a, si_s, di_s, ta_s, ta_d, ts_s, ts_d,
                 g0, g1, g2, g3,
                 i0, i1, i2, i3, r0, r1, r2, r3, s0, s1, s2, s3):
    c = lax.axis_index("c")
    s = lax.axis_index("s")
    wid = s * 2 + c
    nch_total = g0.shape[0] // CHUNK
    extra = nch_total % NWORK
    nch = nch_total // NWORK + jnp.where(wid < extra, 1, 0)
    idx_hbm = (si_a, di_a, si_s, di_s)
    tabs = (ta_s, ta_d, ts_s, ts_d)
    outs = (g0, g1, g2, g3)
    idxv = (i0, i1, i2, i3)
    rowv = (r0, r1, r2, r3)
    sems = (s0, s1, s2, s3)

    def body(i, carry):
        base = (wid + i * NWORK) * CHUNK
        for k in range(4):
            pltpu.sync_copy(idx_hbm[k].at[pl.ds(base, CHUNK)], idxv[k])
        cps = [pltpu.async_copy(tabs[k].at[idxv[k]], rowv[k], sems[k])
               for k in range(4)]
        for cp in cps:
            cp.wait()
        for k in range(4):
            pltpu.sync_copy(rowv[k], outs[k].at[pl.ds(base, CHUNK)])
        return carry

    lax.fori_loop(0, nch, body, 0)


def _gather_tables(si_a, di_a, si_s, di_s, ta_s, ta_d, ts_s, ts_d):
    ea = si_a.shape[0]
    mesh = plsc.VectorSubcoreMesh(core_axis_name="c", subcore_axis_name="s")
    scratch = ([pltpu.VMEM((CHUNK,), jnp.int32)] * 4
               + [pltpu.VMEM((CHUNK, HID), F32)] * 4
               + [pltpu.SemaphoreType.DMA] * 4)
    fn = pl.kernel(
        _gather_body,
        out_type=[jax.ShapeDtypeStruct((ea, HID), F32)] * 4,
        mesh=mesh,
        scratch_types=scratch,
    )
    return fn(si_a, di_a, si_s, di_s, ta_s, ta_d, ts_s, ts_d)


def _edge_body(gs_ref, gd_ref, dis_ref, wd_ref, w2_ref, b2_ref, w3_ref, b3_ref,
               num_ref, den_ref):
    z1 = jnp.tanh(gs_ref[...] + gd_ref[...] + dis_ref[...] * wd_ref[...])
    h2 = jnp.tanh(jnp.dot(z1, w2_ref[...], preferred_element_type=F32)
                  + b2_ref[...])
    o = jnp.dot(h2, w3_ref[...], preferred_element_type=F32) + b3_ref[...]
    el = jnp.exp(o[:, 0:HID])
    num_ref[...] = el * o[:, HID:2 * HID]
    den_ref[...] = el


def _edge_mlp(gs, gd, dis, wd, w2, b2, w3, b3):
    ea = gs.shape[0]
    grid = (ea // EDGE_BLK,)
    row = lambda w: pl.BlockSpec((EDGE_BLK, w), lambda i: (i, 0))
    full = lambda a, b: pl.BlockSpec((a, b), lambda i: (0, 0))
    return pl.pallas_call(
        _edge_body,
        grid=grid,
        in_specs=[row(HID), row(HID), row(1),
                  full(1, HID), full(HID, HID), full(1, HID),
                  full(HID, 2 * HID), full(1, 2 * HID)],
        out_specs=[row(HID), row(HID)],
        out_shape=[jax.ShapeDtypeStruct((ea, HID), F32)] * 2,
    )(gs, gd, dis, wd, w2, b2, w3, b3)


def _scatter_body(didx, num, den, zeros, out_n, out_d, idxv, bufv, acc_sh,
                  sem):
    c = lax.axis_index("c")
    s = lax.axis_index("s")
    nch_total = num.shape[0] // CHUNK
    n_sub = 16
    extra = nch_total % n_sub
    nch = nch_total // n_sub + jnp.where(s < extra, 1, 0)

    @pl.when(s == 0)
    def _():
        pltpu.sync_copy(zeros, acc_sh)

    plsc.subcore_barrier()

    def run(src_hbm):
        def body(i, carry):
            base = (s + i * n_sub) * CHUNK
            pltpu.sync_copy(didx.at[pl.ds(base, CHUNK)], idxv)
            pltpu.sync_copy(src_hbm.at[pl.ds(base, CHUNK)], bufv)
            pltpu.sync_copy(bufv, acc_sh.at[idxv], add=True)
            return carry

        lax.fori_loop(0, nch, body, 0)

    pl.when(c == 0)(lambda: run(num))
    pl.when(c == 1)(lambda: run(den))
    plsc.subcore_barrier()

    @pl.when(s < 10)
    def _():
        r0 = s * 1000
        rows = pl.ds(r0, 1000)
        pl.when(c == 0)(lambda: pltpu.sync_copy(acc_sh.at[rows],
                                                out_n.at[rows]))
        pl.when(c == 1)(lambda: pltpu.sync_copy(acc_sh.at[rows],
                                                out_d.at[rows]))


def _scatter_add(didx, num, den, zeros):
    mesh = plsc.VectorSubcoreMesh(core_axis_name="c", subcore_axis_name="s")
    scratch = [pltpu.VMEM((CHUNK,), jnp.int32),
               pltpu.VMEM((CHUNK, HID), F32),
               pltpu.VMEM_SHARED((N_NODE, HID), F32),
               pltpu.SemaphoreType.DMA]
    fn = pl.kernel(
        _scatter_body,
        out_type=[jax.ShapeDtypeStruct((N_NODE, HID), F32)] * 2,
        mesh=mesh,
        scratch_types=scratch,
    )
    return fn(didx, num, den, zeros)


def _final_body(pos_ref, h_ref, x_ref, numa_ref, dena_ref, nums_ref, dens_ref,
                wp_ref, wh_ref, wsu_ref, wsx_ref, wx2_ref, b1_ref,
                w2_ref, b2_ref, w3_ref, b3_ref, out_ref):
    dena = dena_ref[...]
    dens = dens_ref[...]
    sum_u = jnp.where(dena != 0, numa_ref[...] / dena, 0.0)
    sum_x = jnp.where(dens != 0, nums_ref[...] / dens, 0.0)
    pos = pos_ref[...]
    wp = wp_ref[...]
    t1 = jnp.tanh(
        pos[:, 0:1] * wp[0:1, :] + pos[:, 1:2] * wp[1:2, :]
        + jnp.dot(h_ref[...], wh_ref[...], preferred_element_type=F32)
        + jnp.dot(sum_u, wsu_ref[...], preferred_element_type=F32)
        + jnp.dot(sum_x, wsx_ref[...], preferred_element_type=F32)
        + jnp.dot(x_ref[...], wx2_ref[...], preferred_element_type=F32)
        + b1_ref[...])
    t2 = jnp.tanh(jnp.dot(t1, w2_ref[...], preferred_element_type=F32)
                  + b2_ref[...])
    out_ref[...] = (jnp.dot(t2, w3_ref[...], preferred_element_type=F32)
                    + b3_ref[...])


def _final_mlp(pos_s, h, x, numa, dena, nums, dens, wp, wh, wsu, wsx, wx2, b1,
               w2, b2, w3, b3):
    n = pos_s.shape[0]
    grid = (n // NODE_BLK,)
    row = lambda w: pl.BlockSpec((NODE_BLK, w), lambda i: (i, 0))
    full = lambda a, b: pl.BlockSpec((a, b), lambda i: (0, 0))
    mlp = 64
    return pl.pallas_call(
        _final_body,
        grid=grid,
        in_specs=[row(2), row(HID), row(HID), row(HID), row(HID), row(HID),
                  row(HID),
                  full(2, mlp), full(HID, mlp), full(HID, mlp),
                  full(HID, mlp), full(HID, mlp), full(1, mlp),
                  full(mlp, mlp), full(1, mlp), full(mlp, HID),
                  full(1, HID)],
        out_specs=row(HID),
        out_shape=jax.ShapeDtypeStruct((n, HID), F32),
    )(pos_s, h, x, numa, dena, nums, dens, wp, wh, wsu, wsx, wx2, b1, w2, b2,
      w3, b3)


def _fuse_heads(pa, pb):
    w1 = jnp.concatenate([pa["W1"], pb["W1"]], axis=1)
    b1 = jnp.concatenate([pa["b1"], pb["b1"]])
    z = jnp.zeros_like(pa["W2"])
    w2 = jnp.concatenate([jnp.concatenate([pa["W2"], z], 1),
                          jnp.concatenate([z, pb["W2"]], 1)], 0)
    b2 = jnp.concatenate([pa["b2"], pb["b2"]])
    z3 = jnp.zeros_like(pa["W3"])
    w3 = jnp.concatenate([jnp.concatenate([pa["W3"], z3], 1),
                          jnp.concatenate([z3, pb["W3"]], 1)], 0)
    b3 = jnp.concatenate([pa["b3"], pb["b3"]])
    return w1, b1, w2, b2, w3, b3


def kernel(h, x, u, pos_state, pos_action, dis_a2s, dis_s2s, edge_a2s,
           edge_s2s, params):
    f = HID
    w1u, b1u, w2u, b2u, w3u, b3u = _fuse_heads(params["u2h_logit"],
                                               params["u2h_u"])
    w1x, b1x, w2x, b2x, w3x, b3x = _fuse_heads(params["x2h_logit"],
                                               params["x2h_x"])
    wp = jnp.concatenate([w1u[2:4], w1x[0:2], w1x[2:4]], axis=1)
    wh = jnp.concatenate([w1u[133:261], w1x[5:133], w1x[261:389]], 1)
    wx = jnp.concatenate([w1u[261:389], w1x[133:261], w1x[389:517]], 1)
    bs = jnp.concatenate([b1u, jnp.zeros_like(b1x), b1x]).reshape(1, 3 * f)
    wpa = w1u[0:2]
    wua = w1u[5:133]
    ta_dst, ts_src, ts_dst, ta_src = _node_tables(
        pos_state, h, x, pos_action, u, wp, wh, wx, bs, wpa, wua)

    src_a = edge_a2s[0].astype(jnp.int32)
    dst_a = edge_a2s[1].astype(jnp.int32)
    src_s = edge_s2s[0].astype(jnp.int32)
    dst_s = edge_s2s[1].astype(jnp.int32)

    g_as, g_ad, g_ss, g_sd = _gather_tables(
        src_a, dst_a, src_s, dst_s, ta_src, ta_dst, ts_src, ts_dst)

    num_a, den_a = _edge_mlp(g_as, g_ad, dis_a2s, w1u[4:5], w2u,
                             b2u.reshape(1, 2 * 64), w3u,
                             b3u.reshape(1, 2 * f))
    num_s, den_s = _edge_mlp(g_ss, g_sd, dis_s2s, w1x[4:5], w2x,
                             b2x.reshape(1, 2 * 64), w3x,
                             b3x.reshape(1, 2 * f))

    zeros = jnp.zeros((N_NODE, f), F32)
    numa, dena = _scatter_add(dst_a, num_a, den_a, zeros)
    nums, dens = _scatter_add(dst_s, num_s, den_s, zeros)

    pu = params["h_updater"]
    w1f = pu["W1"]
    return _final_mlp(
        pos_state, h, x, numa, dena, nums, dens,
        w1f[0:2], w1f[2:130], w1f[130:258], w1f[258:386], w1f[386:514],
        pu["b1"].reshape(1, -1), pu["W2"], pu["b2"].reshape(1, -1),
        pu["W3"], pu["b3"].reshape(1, -1))
